# baseline (speedup 1.0000x reference)
import functools

import jax
import jax.numpy as jnp
from jax import lax
from jax.experimental import pallas as pl
from jax.experimental.pallas import tpu as pltpu
from jax.experimental.pallas import tpu_sc as plsc

N = 10000
D = 128
E = 320000
NB = 100

NC = 2
NS = 16
NW = NC * NS
K = 128
K16 = K * 16
NCH0 = 120
NCH1 = 40
NCHMAX = max(NCH0, NCH1)
CH0TOT = NS * NCH0
TOTCH = NS * (NCH0 + NCH1)
TOTE = TOTCH * K
TOTCH_PAD = CH0TOT + (NS - 1) * NCH1 + NCHMAX
TOTE_PAD = TOTCH_PAD * K

AGGR = 10112
RPT = AGGR // NS
DS = 10240
DPT = DS // NS

BLK = 1000
GRID = N // BLK

def _b16(v):
    return jnp.zeros((16,), jnp.int32) + v


def _chunk_base(cid, sid):
    nch = jnp.where(cid == 0, NCH0, NCH1)
    chb = jnp.where(cid == 0, sid * NCH0, CH0TOT + sid * NCH1)
    return nch, chb


def _deg_body(col_hbm, w_hbm, out_hbm, col_v, w_v, zb, deg_sh, sem):
    cid = lax.axis_index("c")
    sid = lax.axis_index("s")
    nch, chb = _chunk_base(cid, sid)

    def zfill(i, _):
        zb[pl.ds(i * 16, 16)] = jnp.zeros((16,), jnp.float32)
        return _
    lax.fori_loop(0, DPT // 16, zfill, None)
    pltpu.sync_copy(zb, deg_sh.at[pl.ds(sid * DPT, DPT)])

    pltpu.sync_copy(col_hbm.at[pl.ds(chb, NCHMAX)], col_v)
    pltpu.sync_copy(w_hbm.at[pl.ds(chb * K, NCHMAX * K)], w_v)
    plsc.subcore_barrier()

    def fire(ci, _):
        pltpu.async_copy(w_v.at[pl.ds(ci * K, K)], deg_sh.at[col_v.at[ci]],
                         sem, add=True)
        return _
    lax.fori_loop(0, nch, fire, None)

    def drain(ci, _):
        pltpu.make_async_copy(w_v.at[pl.ds(ci * K, K)],
                              deg_sh.at[col_v.at[ci]], sem).wait()
        return _
    lax.fori_loop(0, nch, drain, None)

    plsc.subcore_barrier()
    pltpu.sync_copy(deg_sh.at[pl.ds(sid * DPT, DPT)],
                    out_hbm.at[cid, pl.ds(sid * DPT, DPT)])


@functools.cache
def _deg_call():
    return pl.kernel(
        _deg_body,
        out_type=jax.ShapeDtypeStruct((NC, DS), jnp.float32),
        mesh=plsc.VectorSubcoreMesh(core_axis_name="c", subcore_axis_name="s"),
        scratch_types=[
            pltpu.VMEM((NCHMAX, K), jnp.int32),
            pltpu.VMEM((NCHMAX * K,), jnp.float32),
            pltpu.VMEM((DPT,), jnp.float32),
            pltpu.VMEM_SHARED((DS,), jnp.float32),
            pltpu.SemaphoreType.DMA,
        ],
    )


def _agg_body(y_hbm, row_hbm, col_hbm, w16_hbm, out_hbm,
              row_b, col_b, wbs, gs, agg_sh, sems_g, sems_w, sems_c, sems_r):
    cid = lax.axis_index("c")
    sid = lax.axis_index("s")
    nch, chb = _chunk_base(cid, sid)
    g0 = gs[0]

    def zfill(j, _):
        for s in range(8):
            g0[j, pl.ds(s * 16, 16)] = jnp.zeros((16,), jnp.float32)
        return _
    lax.fori_loop(0, K, zfill, None)
    base = sid * RPT
    for kk in range(RPT // K):
        pltpu.async_copy(g0, agg_sh.at[pl.ds(base + kk * K, K)], sems_r[0])
    if RPT % K:
        pltpu.async_copy(g0.at[pl.ds(0, RPT % K)],
                         agg_sh.at[pl.ds(base + (RPT // K) * K, RPT % K)],
                         sems_r[0])
    for kk in range(RPT // K):
        pltpu.make_async_copy(g0, agg_sh.at[pl.ds(base + kk * K, K)],
                              sems_r[0]).wait()
    if RPT % K:
        pltpu.make_async_copy(g0.at[pl.ds(0, RPT % K)],
                              agg_sh.at[pl.ds(base + (RPT // K) * K, RPT % K)],
                              sems_r[0]).wait()

    def gather(ci, b):
        pltpu.async_copy(y_hbm.at[row_b.at[b]], gs[b], sems_g[b])
        pltpu.async_copy(w16_hbm.at[pl.ds((chb + ci) * K16, K16)], wbs[b],
                         sems_w[b])

    pltpu.sync_copy(row_hbm.at[pl.ds(chb * K, K)], row_b.at[0])
    pltpu.sync_copy(row_hbm.at[pl.ds((chb + 1) * K, K)], row_b.at[1])
    gather(0, 0)
    gather(1, 1)
    plsc.subcore_barrier()

    def chunk(ci, b):
        pltpu.async_copy(col_hbm.at[pl.ds((chb + ci) * K, K)], col_b.at[b],
                         sems_c[b])
        pltpu.make_async_copy(y_hbm.at[row_b.at[b]], gs[b], sems_g[b]).wait()
        pltpu.make_async_copy(w16_hbm.at[pl.ds((chb + ci) * K16, K16)],
                              wbs[b], sems_w[b]).wait()

        @pl.when(ci + 2 < nch)
        def _():
            pltpu.async_copy(row_hbm.at[pl.ds((chb + ci + 2) * K, K)],
                             row_b.at[b], sems_r[b])

        g = gs[b]
        wb = wbs[b]

        def scale(j, _):
            wsp = wb[pl.ds(j * 16, 16)]
            for s in range(8):
                sl = pl.ds(s * 16, 16)
                g[j, sl] = g[j, sl] * wsp
            return _
        lax.fori_loop(0, K, scale, None, unroll=8)

        pltpu.make_async_copy(col_hbm.at[pl.ds((chb + ci) * K, K)],
                              col_b.at[b], sems_c[b]).wait()
        pltpu.sync_copy(g, agg_sh.at[col_b.at[b]], add=True)

        @pl.when(ci + 2 < nch)
        def _():
            pltpu.make_async_copy(row_hbm.at[pl.ds((chb + ci + 2) * K, K)],
                                  row_b.at[b], sems_r[b]).wait()
            gather(ci + 2, b)

    def pair(i, _):
        chunk(2 * i, 0)
        chunk(2 * i + 1, 1)
        return _
    lax.fori_loop(0, nch // 2, pair, None)

    plsc.subcore_barrier()
    for kk in range(RPT // K):
        pltpu.async_copy(agg_sh.at[pl.ds(base + kk * K, K)],
                         out_hbm.at[cid, pl.ds(base + kk * K, K)], sems_r[0])
    off = (RPT // K) * K
    if RPT % K:
        pltpu.async_copy(agg_sh.at[pl.ds(base + off, RPT % K)],
                         out_hbm.at[cid, pl.ds(base + off, RPT % K)],
                         sems_r[0])
    for kk in range(RPT // K):
        pltpu.make_async_copy(agg_sh.at[pl.ds(base + kk * K, K)],
                              out_hbm.at[cid, pl.ds(base + kk * K, K)],
                              sems_r[0]).wait()
    if RPT % K:
        pltpu.make_async_copy(agg_sh.at[pl.ds(base + off, RPT % K)],
                              out_hbm.at[cid, pl.ds(base + off, RPT % K)],
                              sems_r[0]).wait()


@functools.cache
def _agg_call():
    return pl.kernel(
        _agg_body,
        out_type=jax.ShapeDtypeStruct((NC, AGGR, D), jnp.float32),
        mesh=plsc.VectorSubcoreMesh(core_axis_name="c", subcore_axis_name="s"),
        scratch_types=[
            pltpu.VMEM((2, K), jnp.int32),
            pltpu.VMEM((2, K), jnp.int32),
            [pltpu.VMEM((K16,), jnp.float32)] * 2,
            [pltpu.VMEM((K, D), jnp.float32)] * 2,
            pltpu.VMEM_SHARED((AGGR, D), jnp.float32),
            [pltpu.SemaphoreType.DMA] * 2,
            [pltpu.SemaphoreType.DMA] * 2,
            [pltpu.SemaphoreType.DMA] * 2,
            [pltpu.SemaphoreType.DMA] * 2,
        ],
    )


def _wx_body(w_ref, out_ref):
    w = w_ref[...]
    out_ref[...] = jnp.repeat(w, 16, axis=1)


_wx = pl.pallas_call(
    _wx_body,
    grid=(TOTCH_PAD // 64,),
    in_specs=[pl.BlockSpec((64, K), lambda i: (i, 0))],
    out_specs=pl.BlockSpec((64, K16), lambda i: (i, 0)),
    out_shape=jax.ShapeDtypeStruct((TOTCH_PAD, K16), jnp.float32),
)


def _tc_pro_body(x0_ref, gw_ref, degt_ref, w0_ref, y0_ref):
    x0 = x0_ref[...]
    gw = gw_ref[...]
    dg = degt_ref[...]
    dinv = lax.rsqrt(dg[:, 0:1] + dg[:, 1:2] + 1.0)
    r = jax.nn.sigmoid(x0 * gw[0:1] + gw[3:4])
    z = jax.nn.sigmoid(x0 * gw[1:2] + gw[4:5])
    n = jnp.tanh(x0 * gw[2:3] + gw[5:6] + r * gw[6:7])
    h = (1.0 - z) * n
    y0_ref[...] = dinv * jnp.dot(h, w0_ref[...],
                                 preferred_element_type=jnp.float32)


_tc_pro = pl.pallas_call(
    _tc_pro_body,
    grid=(GRID,),
    in_specs=[
        pl.BlockSpec((BLK, 1), lambda i: (i, 0)),
        pl.BlockSpec((8, D), lambda i: (0, 0)),
        pl.BlockSpec((BLK, 2), lambda i: (i, 0)),
        pl.BlockSpec((D, D), lambda i: (0, 0)),
    ],
    out_specs=pl.BlockSpec((BLK, D), lambda i: (i, 0)),
    out_shape=jax.ShapeDtypeStruct((N, D), jnp.float32),
)


def _tc_mid_body(agg_ref, y_ref, degt_ref, b_ref, a_ref, wn_ref, out_ref):
    dg = degt_ref[...]
    dinv = lax.rsqrt(dg[:, 0:1] + dg[:, 1:2] + 1.0)
    agg = agg_ref[0] + agg_ref[1]
    out = dinv * (agg + y_ref[...]) + b_ref[...]
    a = a_ref[0, 0]
    h = jnp.where(out >= 0, out, a * out)
    out_ref[...] = dinv * jnp.dot(h, wn_ref[...],
                                  preferred_element_type=jnp.float32)


_tc_mid = pl.pallas_call(
    _tc_mid_body,
    grid=(GRID,),
    in_specs=[
        pl.BlockSpec((NC, BLK, D), lambda i: (0, i, 0)),
        pl.BlockSpec((BLK, D), lambda i: (i, 0)),
        pl.BlockSpec((BLK, 2), lambda i: (i, 0)),
        pl.BlockSpec((1, D), lambda i: (0, 0)),
        pl.BlockSpec((1, 1), lambda i: (0, 0)),
        pl.BlockSpec((D, D), lambda i: (0, 0)),
    ],
    out_specs=pl.BlockSpec((BLK, D), lambda i: (i, 0)),
    out_shape=jax.ShapeDtypeStruct((N, D), jnp.float32),
)


def _tc_epi_body(agg_ref, y_ref, degt_ref, b_ref, a_ref, bat_ref,
                 ne_ref, ge_ref, acc_ref, cacc_ref):
    i = pl.program_id(0)
    dg = degt_ref[...]
    dinv = lax.rsqrt(dg[:, 0:1] + dg[:, 1:2] + 1.0)
    agg = agg_ref[0] + agg_ref[1]
    out = dinv * (agg + y_ref[...]) + b_ref[...]
    a = a_ref[0, 0]
    h = jnp.where(out >= 0, out, a * out)
    ne_ref[...] = h

    bat = bat_ref[0]
    io = lax.broadcasted_iota(jnp.int32, (NB, BLK), 0)
    m = (io == bat).astype(jnp.float32)
    pp = jnp.dot(m, h, preferred_element_type=jnp.float32)
    cp = jnp.sum(m, axis=1, keepdims=True)

    @pl.when(i == 0)
    def _():
        acc_ref[...] = pp
        cacc_ref[...] = cp

    @pl.when(i > 0)
    def _():
        acc_ref[...] += pp
        cacc_ref[...] += cp

    @pl.when(i == pl.num_programs(0) - 1)
    def _():
        ge_ref[...] = acc_ref[...] / jnp.maximum(cacc_ref[...], 1.0)


_tc_epi = pl.pallas_call(
    _tc_epi_body,
    grid=(GRID,),
    in_specs=[
        pl.BlockSpec((NC, BLK, D), lambda i: (0, i, 0)),
        pl.BlockSpec((BLK, D), lambda i: (i, 0)),
        pl.BlockSpec((BLK, 2), lambda i: (i, 0)),
        pl.BlockSpec((1, D), lambda i: (0, 0)),
        pl.BlockSpec((1, 1), lambda i: (0, 0)),
        pl.BlockSpec((1, 1, BLK), lambda i: (i, 0, 0)),
    ],
    out_specs=[
        pl.BlockSpec((BLK, D), lambda i: (i, 0)),
        pl.BlockSpec((NB, D), lambda i: (0, 0)),
    ],
    out_shape=[
        jax.ShapeDtypeStruct((N, D), jnp.float32),
        jax.ShapeDtypeStruct((NB, D), jnp.float32),
    ],
    scratch_shapes=[
        pltpu.VMEM((NB, D), jnp.float32),
        pltpu.VMEM((NB, 1), jnp.float32),
    ],
)


def kernel(x, edge_index, edge_attr, lengths, batch,
           gru_w_ih, gru_w_hh, gru_b_ih, gru_b_hh,
           w_g0, b_g0, p_0, w_g1, b_g1, p_1,
           w_g2, b_g2, p_2, w_g3, b_g3, p_3):
    pad = TOTE_PAD - E
    rowf = jnp.pad(edge_index[0], (0, pad))
    colf = jnp.pad(edge_index[1], (0, pad))
    col2d = colf.reshape(TOTCH_PAD, K)
    wp = jnp.pad(edge_attr, (0, pad))
    w16 = _wx(wp.reshape(TOTCH_PAD, K)).reshape(TOTE_PAD * 16)

    wih = gru_w_ih[:, 0]
    gw = jnp.stack([
        wih[0:D], wih[D:2 * D], wih[2 * D:3 * D],
        gru_b_ih[0:D] + gru_b_hh[0:D],
        gru_b_ih[D:2 * D] + gru_b_hh[D:2 * D],
        gru_b_ih[2 * D:3 * D], gru_b_hh[2 * D:3 * D],
        jnp.zeros((D,), jnp.float32),
    ])

    degp = _deg_call()(col2d, wp)
    degt = jnp.stack([degp[0, :N], degp[1, :N]], axis=1)

    x0 = x[:, 0:1]
    bat = batch.reshape(GRID, 1, BLK)
    ws = [w_g0, w_g1, w_g2, w_g3]
    bs = [b_g0.reshape(1, D), b_g1.reshape(1, D),
          b_g2.reshape(1, D), b_g3.reshape(1, D)]
    ps = [p_0.reshape(1, 1), p_1.reshape(1, 1),
          p_2.reshape(1, 1), p_3.reshape(1, 1)]

    y = _tc_pro(x0, gw, degt, ws[0])
    for l in range(3):
        aggp = _agg_call()(y, rowf, colf, w16)
        y = _tc_mid(aggp, y, degt, bs[l], ps[l], ws[l + 1])
    aggp = _agg_call()(y, rowf, colf, w16)
    node_emb, graph_emb = _tc_epi(aggp, y, degt, bs[3], ps[3], bat)
    return node_emb, graph_emb


# speedup vs baseline: 1.3388x; 1.3388x over previous
import functools

import jax
import jax.numpy as jnp
from jax import lax
from jax.experimental import pallas as pl
from jax.experimental.pallas import tpu as pltpu
from jax.experimental.pallas import tpu_sc as plsc

N = 10000
D = 128
E = 320000
NB = 100

NC = 2
NS = 16
NW = NC * NS
K = 128
K16 = K * 16
NCH0 = 120
NCH1 = 40
NCHMAX = max(NCH0, NCH1)
CH0TOT = NS * NCH0
TOTCH = NS * (NCH0 + NCH1)
TOTE = TOTCH * K
TOTCH_PAD = CH0TOT + (NS - 1) * NCH1 + NCHMAX
TOTE_PAD = TOTCH_PAD * K

AGGR = 10112
RPT = AGGR // NS
DS = 10240
DPT = DS // NS

BLK = 1000
GRID = N // BLK

def _b16(v):
    return jnp.zeros((16,), jnp.int32) + v


def _chunk_base(cid, sid):
    nch = jnp.where(cid == 0, NCH0, NCH1)
    chb = jnp.where(cid == 0, sid * NCH0, CH0TOT + sid * NCH1)
    return nch, chb


def _deg_body(col_hbm, w_hbm, out_hbm, col_v, w_v, zb, deg_sh, sem):
    cid = lax.axis_index("c")
    sid = lax.axis_index("s")
    nch, chb = _chunk_base(cid, sid)

    def zfill(i, _):
        zb[pl.ds(i * 16, 16)] = jnp.zeros((16,), jnp.float32)
        return _
    lax.fori_loop(0, DPT // 16, zfill, None)
    pltpu.sync_copy(zb, deg_sh.at[pl.ds(sid * DPT, DPT)])

    pltpu.sync_copy(col_hbm.at[pl.ds(chb, NCHMAX)], col_v)
    pltpu.sync_copy(w_hbm.at[pl.ds(chb * K, NCHMAX * K)], w_v)
    plsc.subcore_barrier()

    def fire(ci, _):
        pltpu.async_copy(w_v.at[pl.ds(ci * K, K)], deg_sh.at[col_v.at[ci]],
                         sem, add=True)
        return _
    lax.fori_loop(0, nch, fire, None)

    def drain(ci, _):
        pltpu.make_async_copy(w_v.at[pl.ds(ci * K, K)],
                              deg_sh.at[col_v.at[ci]], sem).wait()
        return _
    lax.fori_loop(0, nch, drain, None)

    plsc.subcore_barrier()
    pltpu.sync_copy(deg_sh.at[pl.ds(sid * DPT, DPT)],
                    out_hbm.at[cid, pl.ds(sid * DPT, DPT)])


@functools.cache
def _deg_call():
    return pl.kernel(
        _deg_body,
        out_type=jax.ShapeDtypeStruct((NC, DS), jnp.float32),
        mesh=plsc.VectorSubcoreMesh(core_axis_name="c", subcore_axis_name="s"),
        scratch_types=[
            pltpu.VMEM((NCHMAX, K), jnp.int32),
            pltpu.VMEM((NCHMAX * K,), jnp.float32),
            pltpu.VMEM((DPT,), jnp.float32),
            pltpu.VMEM_SHARED((DS,), jnp.float32),
            pltpu.SemaphoreType.DMA,
        ],
    )


def _agg_body(y_hbm, row_hbm, col_hbm, w16_hbm, out_hbm,
              row_b, col_b, wbs, gs, agg_sh, sems_g, sems_w, sems_c, sems_r):
    cid = lax.axis_index("c")
    sid = lax.axis_index("s")
    nch, chb = _chunk_base(cid, sid)
    g0 = gs[0]

    def zfill(j, _):
        for s in range(8):
            g0[j, pl.ds(s * 16, 16)] = jnp.zeros((16,), jnp.float32)
        return _
    lax.fori_loop(0, K, zfill, None)
    base = sid * RPT
    for kk in range(RPT // K):
        pltpu.async_copy(g0, agg_sh.at[pl.ds(base + kk * K, K)], sems_r[0])
    if RPT % K:
        pltpu.async_copy(g0.at[pl.ds(0, RPT % K)],
                         agg_sh.at[pl.ds(base + (RPT // K) * K, RPT % K)],
                         sems_r[0])
    for kk in range(RPT // K):
        pltpu.make_async_copy(g0, agg_sh.at[pl.ds(base + kk * K, K)],
                              sems_r[0]).wait()
    if RPT % K:
        pltpu.make_async_copy(g0.at[pl.ds(0, RPT % K)],
                              agg_sh.at[pl.ds(base + (RPT // K) * K, RPT % K)],
                              sems_r[0]).wait()

    def gather(ci, b):
        pltpu.async_copy(y_hbm.at[row_b.at[b]], gs[b], sems_g[b])
        pltpu.async_copy(w16_hbm.at[pl.ds((chb + ci) * K16, K16)], wbs[b],
                         sems_w[b])

    pltpu.sync_copy(row_hbm.at[pl.ds(chb * K, K)], row_b.at[0])
    pltpu.sync_copy(row_hbm.at[pl.ds((chb + 1) * K, K)], row_b.at[1])
    gather(0, 0)
    gather(1, 1)
    plsc.subcore_barrier()

    def chunk(ci, b):
        pltpu.async_copy(col_hbm.at[pl.ds((chb + ci) * K, K)], col_b.at[b],
                         sems_c[b])
        pltpu.make_async_copy(y_hbm.at[row_b.at[b]], gs[b], sems_g[b]).wait()
        pltpu.make_async_copy(w16_hbm.at[pl.ds((chb + ci) * K16, K16)],
                              wbs[b], sems_w[b]).wait()

        @pl.when(ci + 2 < nch)
        def _():
            pltpu.async_copy(row_hbm.at[pl.ds((chb + ci + 2) * K, K)],
                             row_b.at[b], sems_r[b])

        g = gs[b]
        wb = wbs[b]

        def scale(j, _):
            wsp = wb[pl.ds(j * 16, 16)]
            for s in range(8):
                sl = pl.ds(s * 16, 16)
                g[j, sl] = g[j, sl] * wsp
            return _
        lax.fori_loop(0, K, scale, None, unroll=4)

        pltpu.make_async_copy(col_hbm.at[pl.ds((chb + ci) * K, K)],
                              col_b.at[b], sems_c[b]).wait()
        pltpu.sync_copy(g, agg_sh.at[col_b.at[b]], add=True)

        @pl.when(ci + 2 < nch)
        def _():
            pltpu.make_async_copy(row_hbm.at[pl.ds((chb + ci + 2) * K, K)],
                                  row_b.at[b], sems_r[b]).wait()
            gather(ci + 2, b)

    def pair(i, _):
        chunk(2 * i, 0)
        chunk(2 * i + 1, 1)
        return _
    lax.fori_loop(0, nch // 2, pair, None)

    plsc.subcore_barrier()
    for kk in range(RPT // K):
        pltpu.async_copy(agg_sh.at[pl.ds(base + kk * K, K)],
                         out_hbm.at[cid, pl.ds(base + kk * K, K)], sems_r[0])
    off = (RPT // K) * K
    if RPT % K:
        pltpu.async_copy(agg_sh.at[pl.ds(base + off, RPT % K)],
                         out_hbm.at[cid, pl.ds(base + off, RPT % K)],
                         sems_r[0])
    for kk in range(RPT // K):
        pltpu.make_async_copy(agg_sh.at[pl.ds(base + kk * K, K)],
                              out_hbm.at[cid, pl.ds(base + kk * K, K)],
                              sems_r[0]).wait()
    if RPT % K:
        pltpu.make_async_copy(agg_sh.at[pl.ds(base + off, RPT % K)],
                              out_hbm.at[cid, pl.ds(base + off, RPT % K)],
                              sems_r[0]).wait()


@functools.cache
def _agg_call():
    return pl.kernel(
        _agg_body,
        out_type=jax.ShapeDtypeStruct((NC, AGGR, D), jnp.float32),
        mesh=plsc.VectorSubcoreMesh(core_axis_name="c", subcore_axis_name="s"),
        scratch_types=[
            pltpu.VMEM((2, K), jnp.int32),
            pltpu.VMEM((2, K), jnp.int32),
            [pltpu.VMEM((K16,), jnp.float32)] * 2,
            [pltpu.VMEM((K, D), jnp.float32)] * 2,
            pltpu.VMEM_SHARED((AGGR, D), jnp.float32),
            [pltpu.SemaphoreType.DMA] * 2,
            [pltpu.SemaphoreType.DMA] * 2,
            [pltpu.SemaphoreType.DMA] * 2,
            [pltpu.SemaphoreType.DMA] * 2,
        ],
    )


def _wx_body(w_ref, out_ref):
    w = w_ref[...]
    out_ref[...] = jnp.repeat(w, 16, axis=1)


_wx = pl.pallas_call(
    _wx_body,
    grid=(TOTCH_PAD // 64,),
    in_specs=[pl.BlockSpec((64, K), lambda i: (i, 0))],
    out_specs=pl.BlockSpec((64, K16), lambda i: (i, 0)),
    out_shape=jax.ShapeDtypeStruct((TOTCH_PAD, K16), jnp.float32),
)


def _tc_pro_body(x0_ref, gw_ref, degt_ref, w0_ref, y0_ref):
    x0 = x0_ref[...]
    gw = gw_ref[...]
    dg = degt_ref[...]
    dinv = lax.rsqrt(dg[:, 0:1] + dg[:, 1:2] + 1.0)
    r = jax.nn.sigmoid(x0 * gw[0:1] + gw[3:4])
    z = jax.nn.sigmoid(x0 * gw[1:2] + gw[4:5])
    n = jnp.tanh(x0 * gw[2:3] + gw[5:6] + r * gw[6:7])
    h = (1.0 - z) * n
    y0_ref[...] = dinv * jnp.dot(h, w0_ref[...],
                                 preferred_element_type=jnp.float32)


_tc_pro = pl.pallas_call(
    _tc_pro_body,
    grid=(GRID,),
    in_specs=[
        pl.BlockSpec((BLK, 1), lambda i: (i, 0)),
        pl.BlockSpec((8, D), lambda i: (0, 0)),
        pl.BlockSpec((BLK, 2), lambda i: (i, 0)),
        pl.BlockSpec((D, D), lambda i: (0, 0)),
    ],
    out_specs=pl.BlockSpec((BLK, D), lambda i: (i, 0)),
    out_shape=jax.ShapeDtypeStruct((N, D), jnp.float32),
)


def _tc_mid_body(agg_ref, y_ref, degt_ref, b_ref, a_ref, wn_ref, out_ref):
    dg = degt_ref[...]
    dinv = lax.rsqrt(dg[:, 0:1] + dg[:, 1:2] + 1.0)
    agg = agg_ref[0] + agg_ref[1]
    out = dinv * (agg + y_ref[...]) + b_ref[...]
    a = a_ref[0, 0]
    h = jnp.where(out >= 0, out, a * out)
    out_ref[...] = dinv * jnp.dot(h, wn_ref[...],
                                  preferred_element_type=jnp.float32)


_tc_mid = pl.pallas_call(
    _tc_mid_body,
    grid=(GRID,),
    in_specs=[
        pl.BlockSpec((NC, BLK, D), lambda i: (0, i, 0)),
        pl.BlockSpec((BLK, D), lambda i: (i, 0)),
        pl.BlockSpec((BLK, 2), lambda i: (i, 0)),
        pl.BlockSpec((1, D), lambda i: (0, 0)),
        pl.BlockSpec((1, 1), lambda i: (0, 0)),
        pl.BlockSpec((D, D), lambda i: (0, 0)),
    ],
    out_specs=pl.BlockSpec((BLK, D), lambda i: (i, 0)),
    out_shape=jax.ShapeDtypeStruct((N, D), jnp.float32),
)


def _tc_epi_body(agg_ref, y_ref, degt_ref, b_ref, a_ref, bat_ref,
                 ne_ref, ge_ref, acc_ref, cacc_ref):
    i = pl.program_id(0)
    dg = degt_ref[...]
    dinv = lax.rsqrt(dg[:, 0:1] + dg[:, 1:2] + 1.0)
    agg = agg_ref[0] + agg_ref[1]
    out = dinv * (agg + y_ref[...]) + b_ref[...]
    a = a_ref[0, 0]
    h = jnp.where(out >= 0, out, a * out)
    ne_ref[...] = h

    bat = bat_ref[0]
    io = lax.broadcasted_iota(jnp.int32, (NB, BLK), 0)
    m = (io == bat).astype(jnp.float32)
    pp = jnp.dot(m, h, preferred_element_type=jnp.float32)
    cp = jnp.sum(m, axis=1, keepdims=True)

    @pl.when(i == 0)
    def _():
        acc_ref[...] = pp
        cacc_ref[...] = cp

    @pl.when(i > 0)
    def _():
        acc_ref[...] += pp
        cacc_ref[...] += cp

    @pl.when(i == pl.num_programs(0) - 1)
    def _():
        ge_ref[...] = acc_ref[...] / jnp.maximum(cacc_ref[...], 1.0)


_tc_epi = pl.pallas_call(
    _tc_epi_body,
    grid=(GRID,),
    in_specs=[
        pl.BlockSpec((NC, BLK, D), lambda i: (0, i, 0)),
        pl.BlockSpec((BLK, D), lambda i: (i, 0)),
        pl.BlockSpec((BLK, 2), lambda i: (i, 0)),
        pl.BlockSpec((1, D), lambda i: (0, 0)),
        pl.BlockSpec((1, 1), lambda i: (0, 0)),
        pl.BlockSpec((1, 1, BLK), lambda i: (i, 0, 0)),
    ],
    out_specs=[
        pl.BlockSpec((BLK, D), lambda i: (i, 0)),
        pl.BlockSpec((NB, D), lambda i: (0, 0)),
    ],
    out_shape=[
        jax.ShapeDtypeStruct((N, D), jnp.float32),
        jax.ShapeDtypeStruct((NB, D), jnp.float32),
    ],
    scratch_shapes=[
        pltpu.VMEM((NB, D), jnp.float32),
        pltpu.VMEM((NB, 1), jnp.float32),
    ],
)


def kernel(x, edge_index, edge_attr, lengths, batch,
           gru_w_ih, gru_w_hh, gru_b_ih, gru_b_hh,
           w_g0, b_g0, p_0, w_g1, b_g1, p_1,
           w_g2, b_g2, p_2, w_g3, b_g3, p_3):
    pad = TOTE_PAD - E
    rowf = jnp.pad(edge_index[0], (0, pad))
    colf = jnp.pad(edge_index[1], (0, pad))
    col2d = colf.reshape(TOTCH_PAD, K)
    wp = jnp.pad(edge_attr, (0, pad))
    w16 = _wx(wp.reshape(TOTCH_PAD, K)).reshape(TOTE_PAD * 16)

    wih = gru_w_ih[:, 0]
    gw = jnp.stack([
        wih[0:D], wih[D:2 * D], wih[2 * D:3 * D],
        gru_b_ih[0:D] + gru_b_hh[0:D],
        gru_b_ih[D:2 * D] + gru_b_hh[D:2 * D],
        gru_b_ih[2 * D:3 * D], gru_b_hh[2 * D:3 * D],
        jnp.zeros((D,), jnp.float32),
    ])

    degp = _deg_call()(col2d, wp)
    degt = jnp.stack([degp[0, :N], degp[1, :N]], axis=1)

    x0 = x[:, 0:1]
    bat = batch.reshape(GRID, 1, BLK)
    ws = [w_g0, w_g1, w_g2, w_g3]
    bs = [b_g0.reshape(1, D), b_g1.reshape(1, D),
          b_g2.reshape(1, D), b_g3.reshape(1, D)]
    ps = [p_0.reshape(1, 1), p_1.reshape(1, 1),
          p_2.reshape(1, 1), p_3.reshape(1, 1)]

    y = _tc_pro(x0, gw, degt, ws[0])
    for l in range(3):
        aggp = _agg_call()(y, rowf, colf, w16)
        y = _tc_mid(aggp, y, degt, bs[l], ps[l], ws[l + 1])
    aggp = _agg_call()(y, rowf, colf, w16)
    node_emb, graph_emb = _tc_epi(aggp, y, degt, bs[3], ps[3], bat)
    return node_emb, graph_emb
